# baseline (speedup 1.0000x reference)
import functools

import jax
import jax.numpy as jnp
from jax import lax
from jax.experimental import pallas as pl
from jax.experimental.pallas import tpu as pltpu
from jax.experimental.pallas import tpu_sc as plsc

NN = 10000
EE = 320000
N_PAD = 10240
E_PAD = 327680
K = 128
TILES = 16
EPT_F = E_PAD // 16
CHUNKS_F = EPT_F // K
EPT_E = E_PAD // 32
CHUNKS_E = EPT_E // K
ROWS_PT = N_PAD // TILES
G = 16
TN = 1280
GRID_N = N_PAD // TN

_MESH = dict(core_axis_name="c", subcore_axis_name="s")
_DOT = dict(precision=lax.Precision.DEFAULT, preferred_element_type=jnp.float32)


def _sc_degrees(src2, dst2, zcol):
    @functools.partial(
        pl.kernel,
        mesh=plsc.VectorSubcoreMesh(**_MESH),
        out_type=(jax.ShapeDtypeStruct((N_PAD,), jnp.float32),
                  jax.ShapeDtypeStruct((N_PAD,), jnp.float32)),
        scratch_types=[pltpu.VMEM((CHUNKS_F, K), jnp.int32),
                       pltpu.VMEM((K,), jnp.float32),
                       pltpu.VMEM_SHARED((N_PAD,), jnp.float32),
                       pltpu.SemaphoreType.DMA],
    )
    def deg_kernel(src_hbm, dst_hbm, z_hbm, dsrc_hbm, ddst_hbm,
                   idx_all, ones_v, acc_sh, ssem):
        cid = lax.axis_index("c")
        sid = lax.axis_index("s")
        for i in range(K // 16):
            ones_v[pl.ds(i * 16, 16)] = jnp.ones((16,), jnp.float32)
        sl = pl.ds(sid * ROWS_PT, ROWS_PT)
        pltpu.sync_copy(z_hbm.at[sl], acc_sh.at[sl])

        def run(idx_hbm, out_hbm):
            pltpu.sync_copy(idx_hbm.at[pl.ds(sid * CHUNKS_F, CHUNKS_F)],
                            idx_all)
            plsc.subcore_barrier()

            def body(g, carry):
                for u in range(8):
                    pltpu.make_async_copy(
                        ones_v, acc_sh.at[idx_all.at[g * 8 + u]],
                        ssem).start(add=True)
                for u in range(8):
                    pltpu.make_async_copy(
                        ones_v, acc_sh.at[idx_all.at[g * 8 + u]],
                        ssem).wait()
                return carry
            lax.fori_loop(0, CHUNKS_F // 8, body, 0)
            plsc.subcore_barrier()
            pltpu.sync_copy(acc_sh.at[sl], out_hbm.at[sl])

        @pl.when(cid == 0)
        def _():
            run(src_hbm, dsrc_hbm)

        @pl.when(cid == 1)
        def _():
            run(dst_hbm, ddst_hbm)

    return deg_kernel(src2, dst2, zcol)


def _sc_aggregate(x0, x1, srcp, dstp, zhalf, edge_split):
    @functools.partial(
        pl.kernel,
        mesh=plsc.VectorSubcoreMesh(**_MESH),
        out_type=(jax.ShapeDtypeStruct((N_PAD, 128), jnp.float32),
                  jax.ShapeDtypeStruct((N_PAD, 128), jnp.float32)),
        scratch_types=[pltpu.VMEM((2 * G, K), jnp.int32),
                       pltpu.VMEM((2 * G, K), jnp.int32),
                       pltpu.VMEM((K, 128), jnp.float32),
                       pltpu.VMEM((K, 128), jnp.float32),
                       pltpu.VMEM_SHARED((N_PAD, 128), jnp.float32),
                       pltpu.SemaphoreType.DMA,
                       pltpu.SemaphoreType.DMA,
                       pltpu.SemaphoreType.DMA,
                       pltpu.SemaphoreType.DMA,
                       pltpu.SemaphoreType.DMA],
    )
    def agg_kernel(x0_hbm, x1_hbm, src_hbm, dst_hbm, z_hbm, a0_hbm, a1_hbm,
                   src_g, dst_g, rows0, rows1, acc_sh, g0, g1, s0, s1, isem):
        cid = lax.axis_index("c")
        sid = lax.axis_index("s")
        sl = pl.ds(sid * ROWS_PT, ROWS_PT)

        def run(x_hbm, a_hbm):
            if edge_split:
                cstart = (cid * TILES + sid) * CHUNKS_E
                chunks = CHUNKS_E
            else:
                cstart = sid * CHUNKS_F
                chunks = CHUNKS_F
            groups = chunks // G

            def refill(g):
                gb = cstart + g * G
                hs = pl.ds((g % 2) * G, G)
                pltpu.make_async_copy(
                    src_hbm.at[pl.ds(gb, G)], src_g.at[hs], isem).start()
                pltpu.make_async_copy(
                    dst_hbm.at[pl.ds(gb, G)], dst_g.at[hs], isem).start()

            def refill_wait():
                pltpu.make_async_copy(
                    src_hbm.at[pl.ds(0, G)], src_g.at[pl.ds(0, G)],
                    isem).wait()
                pltpu.make_async_copy(
                    dst_hbm.at[pl.ds(0, G)], dst_g.at[pl.ds(0, G)],
                    isem).wait()

            refill(0)
            pltpu.sync_copy(z_hbm, rows0)
            for r in range(ROWS_PT // K):
                pltpu.make_async_copy(
                    rows0, acc_sh.at[pl.ds(sid * ROWS_PT + r * K, K)],
                    s0).start()
            for r in range(ROWS_PT // K):
                pltpu.make_async_copy(
                    rows0, acc_sh.at[pl.ds(sid * ROWS_PT + r * K, K)],
                    s0).wait()
            refill_wait()
            pltpu.make_async_copy(x_hbm.at[src_g.at[0]], rows0, g0).start()
            pltpu.make_async_copy(x_hbm.at[src_g.at[1]], rows1, g1).start()
            plsc.subcore_barrier()

            def drain_fire(j, rows_v, gsem, ssem):
                pltpu.make_async_copy(
                    x_hbm.at[src_g.at[j & (2 * G - 1)]], rows_v, gsem).wait()
                pltpu.make_async_copy(
                    rows_v, acc_sh.at[dst_g.at[j & (2 * G - 1)]],
                    ssem).start(add=True)

            def settle(j, rows_v, gsem, ssem):
                pltpu.make_async_copy(
                    rows_v, acc_sh.at[dst_g.at[j & (2 * G - 1)]],
                    ssem).wait()

                @pl.when(j + 2 < chunks)
                def _():
                    pltpu.make_async_copy(
                        x_hbm.at[src_g.at[(j + 2) & (2 * G - 1)]],
                        rows_v, gsem).start()

            def pair(p, carry):
                j = 2 * p
                g = p // (G // 2)

                @pl.when((p % (G // 2) == 0) & (g + 1 < groups))
                def _():
                    refill(g + 1)
                drain_fire(j, rows0, g0, s0)
                drain_fire(j + 1, rows1, g1, s1)
                settle(j, rows0, g0, s0)
                settle(j + 1, rows1, g1, s1)

                @pl.when((p % (G // 2) == G // 2 - 2) & (g + 1 < groups))
                def _():
                    refill_wait()
                return carry
            lax.fori_loop(0, chunks // 2, pair, 0)
            plsc.subcore_barrier()
            for r in range(ROWS_PT // K):
                pltpu.make_async_copy(
                    acc_sh.at[pl.ds(sid * ROWS_PT + r * K, K)],
                    a_hbm.at[pl.ds(sid * ROWS_PT + r * K, K)], s0).start()
            for r in range(ROWS_PT // K):
                pltpu.make_async_copy(
                    acc_sh.at[pl.ds(sid * ROWS_PT + r * K, K)],
                    a_hbm.at[pl.ds(sid * ROWS_PT + r * K, K)], s0).wait()

        @pl.when(cid == 0)
        def _():
            run(x0_hbm, a0_hbm)

        @pl.when(cid == 1)
        def _():
            run(x1_hbm, a1_hbm)

    return agg_kernel(x0, x1, srcp, dstp, zhalf)


def _tc_prep(hp, dsrc, ddst):
    def body(h_ref, ds_ref, dd_ref, rin_ref, rout_ref, xs_ref):
        i = pl.program_id(0)
        rows = i * TN + lax.broadcasted_iota(jnp.int32, (TN, 1), 0)
        valid = (rows < NN).astype(jnp.float32)
        rout = lax.rsqrt(jnp.maximum(ds_ref[...], 1.0)) * valid
        rin = lax.rsqrt(jnp.maximum(dd_ref[...], 1.0)) * valid
        rin_ref[...] = rin
        rout_ref[...] = rout
        xs_ref[...] = h_ref[...] * rout

    f32 = jnp.float32
    return pl.pallas_call(
        body,
        grid=(GRID_N,),
        in_specs=[pl.BlockSpec((TN, 128), lambda i: (i, 0)),
                  pl.BlockSpec((TN, 1), lambda i: (i, 0)),
                  pl.BlockSpec((TN, 1), lambda i: (i, 0))],
        out_specs=[pl.BlockSpec((TN, 1), lambda i: (i, 0)),
                   pl.BlockSpec((TN, 1), lambda i: (i, 0)),
                   pl.BlockSpec((TN, 128), lambda i: (i, 0))],
        out_shape=[jax.ShapeDtypeStruct((N_PAD, 1), f32),
                   jax.ShapeDtypeStruct((N_PAD, 1), f32),
                   jax.ShapeDtypeStruct((N_PAD, 128), f32)],
    )(hp, dsrc, ddst)


def _tc_layer(a0, a1, w0, w1, b, rin, rout):
    def body(a0_ref, a1_ref, w0_ref, w1_ref, b_ref, rin_ref, rout_ref,
             y0_ref, y1_ref):
        t = lax.dot_general(a0_ref[...], w0_ref[...],
                            (((1,), (0,)), ((), ())), **_DOT)
        t += lax.dot_general(a1_ref[...], w1_ref[...],
                             (((1,), (0,)), ((), ())), **_DOT)
        t = jnp.maximum(t * rin_ref[...] + b_ref[...], 0.0) * rout_ref[...]
        y0_ref[...] = t[:, :128]
        y1_ref[...] = t[:, 128:]

    f32 = jnp.float32
    return pl.pallas_call(
        body,
        grid=(GRID_N,),
        in_specs=[pl.BlockSpec((TN, 128), lambda i: (i, 0)),
                  pl.BlockSpec((TN, 128), lambda i: (i, 0)),
                  pl.BlockSpec((128, 256), lambda i: (0, 0)),
                  pl.BlockSpec((128, 256), lambda i: (0, 0)),
                  pl.BlockSpec((1, 256), lambda i: (0, 0)),
                  pl.BlockSpec((TN, 1), lambda i: (i, 0)),
                  pl.BlockSpec((TN, 1), lambda i: (i, 0))],
        out_specs=[pl.BlockSpec((TN, 128), lambda i: (i, 0)),
                   pl.BlockSpec((TN, 128), lambda i: (i, 0))],
        out_shape=[jax.ShapeDtypeStruct((N_PAD, 128), f32),
                   jax.ShapeDtypeStruct((N_PAD, 128), f32)],
    )(a0, a1, w0, w1, b, rin, rout)


def _tc_final(a0, a1, w0, w1, b, rin, wc, bc):
    def body(a0_ref, a1_ref, w0_ref, w1_ref, b_ref, rin_ref, wc_ref, bc_ref,
             o_ref):
        i = pl.program_id(0)
        t = lax.dot_general(a0_ref[...], w0_ref[...],
                            (((1,), (0,)), ((), ())), **_DOT)
        t += lax.dot_general(a1_ref[...], w1_ref[...],
                             (((1,), (0,)), ((), ())), **_DOT)
        t = jnp.maximum(t * rin_ref[...] + b_ref[...], 0.0)
        rows = i * TN + lax.broadcasted_iota(jnp.int32, (TN, 1), 0)
        t = jnp.where(rows < NN, t, 0.0)
        cs = jnp.sum(t, axis=0, keepdims=True)
        part = lax.dot_general(cs, wc_ref[...], (((1,), (0,)), ((), ())),
                               **_DOT)

        @pl.when(i == 0)
        def _():
            o_ref[...] = part + bc_ref[...]

        @pl.when(i > 0)
        def _():
            o_ref[...] = o_ref[...] + part

    return pl.pallas_call(
        body,
        grid=(GRID_N,),
        in_specs=[pl.BlockSpec((TN, 128), lambda i: (i, 0)),
                  pl.BlockSpec((TN, 128), lambda i: (i, 0)),
                  pl.BlockSpec((128, 256), lambda i: (0, 0)),
                  pl.BlockSpec((128, 256), lambda i: (0, 0)),
                  pl.BlockSpec((1, 256), lambda i: (0, 0)),
                  pl.BlockSpec((TN, 1), lambda i: (i, 0)),
                  pl.BlockSpec((256, 10), lambda i: (0, 0)),
                  pl.BlockSpec((1, 10), lambda i: (0, 0))],
        out_specs=pl.BlockSpec((1, 10), lambda i: (0, 0)),
        out_shape=jax.ShapeDtypeStruct((1, 10), jnp.float32),
    )(a0, a1, w0, w1, b, rin, wc, bc)


def kernel(h, edge_index, W1, b1, W2, b2, W3, b3, W4, b4, W5, b5, Wc, bc):
    f32 = jnp.float32
    src = edge_index[0]
    dst = edge_index[1]
    pad_idx = (NN + (jnp.arange(E_PAD - EE, dtype=jnp.int32) % 8))
    srcp = jnp.concatenate([src, pad_idx]).reshape(E_PAD // K, K)
    dstp = jnp.concatenate([dst, pad_idx]).reshape(E_PAD // K, K)
    hp = jnp.pad(h, ((0, N_PAD - NN), (0, 0)))

    zcol = jnp.zeros((N_PAD,), f32)
    z128 = jnp.zeros((K, 128), f32)

    deg_src, deg_dst = _sc_degrees(srcp, dstp, zcol)
    rin, rout, xs = _tc_prep(hp, deg_src.reshape(N_PAD, 1),
                             deg_dst.reshape(N_PAD, 1))

    a0, a1 = _sc_aggregate(xs, xs, srcp, dstp, z128, edge_split=True)
    x0, x1 = _tc_layer(a0, a1, W1, W1, b1.reshape(1, 256), rin, rout)
    for (W, b) in ((W2, b2), (W3, b3), (W4, b4)):
        a0, a1 = _sc_aggregate(x0, x1, srcp, dstp, z128, edge_split=False)
        x0, x1 = _tc_layer(a0, a1, W[:128], W[128:], b.reshape(1, 256),
                           rin, rout)
    a0, a1 = _sc_aggregate(x0, x1, srcp, dstp, z128, edge_split=False)
    return _tc_final(a0, a1, W5[:128], W5[128:], b5.reshape(1, 256),
                     rin, Wc, bc.reshape(1, 10))


# speedup vs baseline: 1.2499x; 1.2499x over previous
import functools

import jax
import jax.numpy as jnp
from jax import lax
from jax.experimental import pallas as pl
from jax.experimental.pallas import tpu as pltpu
from jax.experimental.pallas import tpu_sc as plsc

NN = 10000
EE = 320000
N_PAD = 10240
E_PAD = 327680
K = 128
TILES = 16
EPT_F = E_PAD // 16
CHUNKS_F = EPT_F // K
EPT_E = E_PAD // 32
CHUNKS_E = EPT_E // K
ROWS_PT = N_PAD // TILES
G = 16
TN = 1280
GRID_N = N_PAD // TN

_MESH = dict(core_axis_name="c", subcore_axis_name="s")
_DOT = dict(precision=lax.Precision.DEFAULT, preferred_element_type=jnp.float32)


def _sc_degrees(src2, dst2, zcol):
    @functools.partial(
        pl.kernel,
        mesh=plsc.VectorSubcoreMesh(**_MESH),
        out_type=(jax.ShapeDtypeStruct((N_PAD,), jnp.float32),
                  jax.ShapeDtypeStruct((N_PAD,), jnp.float32)),
        scratch_types=[pltpu.VMEM((CHUNKS_F, K), jnp.int32),
                       pltpu.VMEM((K,), jnp.float32),
                       pltpu.VMEM_SHARED((N_PAD,), jnp.float32),
                       pltpu.SemaphoreType.DMA],
    )
    def deg_kernel(src_hbm, dst_hbm, z_hbm, dsrc_hbm, ddst_hbm,
                   idx_all, ones_v, acc_sh, ssem):
        cid = lax.axis_index("c")
        sid = lax.axis_index("s")
        for i in range(K // 16):
            ones_v[pl.ds(i * 16, 16)] = jnp.ones((16,), jnp.float32)
        sl = pl.ds(sid * ROWS_PT, ROWS_PT)
        pltpu.sync_copy(z_hbm.at[sl], acc_sh.at[sl])

        def run(idx_hbm, out_hbm):
            pltpu.sync_copy(idx_hbm.at[pl.ds(sid * CHUNKS_F, CHUNKS_F)],
                            idx_all)
            plsc.subcore_barrier()

            def body(g, carry):
                for u in range(8):
                    pltpu.make_async_copy(
                        ones_v, acc_sh.at[idx_all.at[g * 8 + u]],
                        ssem).start(add=True)
                for u in range(8):
                    pltpu.make_async_copy(
                        ones_v, acc_sh.at[idx_all.at[g * 8 + u]],
                        ssem).wait()
                return carry
            lax.fori_loop(0, CHUNKS_F // 8, body, 0)
            plsc.subcore_barrier()
            pltpu.sync_copy(acc_sh.at[sl], out_hbm.at[sl])

        @pl.when(cid == 0)
        def _():
            run(src_hbm, dsrc_hbm)

        @pl.when(cid == 1)
        def _():
            run(dst_hbm, ddst_hbm)

    return deg_kernel(src2, dst2, zcol)


def _sc_aggregate(x0, x1, srcp, dstp, zhalf, edge_split):
    @functools.partial(
        pl.kernel,
        mesh=plsc.VectorSubcoreMesh(**_MESH),
        out_type=(jax.ShapeDtypeStruct((N_PAD, 128), jnp.float32),
                  jax.ShapeDtypeStruct((N_PAD, 128), jnp.float32)),
        scratch_types=[pltpu.VMEM((2 * G, K), jnp.int32),
                       pltpu.VMEM((2 * G, K), jnp.int32),
                       pltpu.VMEM((K, 128), jnp.float32),
                       pltpu.VMEM((K, 128), jnp.float32),
                       pltpu.VMEM_SHARED((N_PAD, 128), jnp.float32),
                       pltpu.SemaphoreType.DMA,
                       pltpu.SemaphoreType.DMA,
                       pltpu.SemaphoreType.DMA,
                       pltpu.SemaphoreType.DMA,
                       pltpu.SemaphoreType.DMA],
    )
    def agg_kernel(x0_hbm, x1_hbm, src_hbm, dst_hbm, z_hbm, a0_hbm, a1_hbm,
                   src_g, dst_g, rows0, rows1, acc_sh, g0, g1, s0, s1, isem):
        cid = lax.axis_index("c")
        sid = lax.axis_index("s")
        sl = pl.ds(sid * ROWS_PT, ROWS_PT)

        def run(x_hbm, a_hbm):
            if edge_split:
                cstart = (cid * TILES + sid) * CHUNKS_E
                chunks = CHUNKS_E
            else:
                cstart = sid * CHUNKS_F
                chunks = CHUNKS_F
            groups = chunks // G

            def refill(g):
                gb = cstart + g * G
                hs = pl.ds((g % 2) * G, G)
                pltpu.make_async_copy(
                    src_hbm.at[pl.ds(gb, G)], src_g.at[hs], isem).start()
                pltpu.make_async_copy(
                    dst_hbm.at[pl.ds(gb, G)], dst_g.at[hs], isem).start()

            def refill_wait():
                pltpu.make_async_copy(
                    src_hbm.at[pl.ds(0, G)], src_g.at[pl.ds(0, G)],
                    isem).wait()
                pltpu.make_async_copy(
                    dst_hbm.at[pl.ds(0, G)], dst_g.at[pl.ds(0, G)],
                    isem).wait()

            refill(0)
            pltpu.sync_copy(z_hbm, rows0)
            for r in range(ROWS_PT // K):
                pltpu.make_async_copy(
                    rows0, acc_sh.at[pl.ds(sid * ROWS_PT + r * K, K)],
                    s0).start()
            for r in range(ROWS_PT // K):
                pltpu.make_async_copy(
                    rows0, acc_sh.at[pl.ds(sid * ROWS_PT + r * K, K)],
                    s0).wait()
            refill_wait()
            pltpu.make_async_copy(x_hbm.at[src_g.at[0]], rows0, g0).start()
            pltpu.make_async_copy(x_hbm.at[src_g.at[1]], rows1, g1).start()
            plsc.subcore_barrier()

            def stage(j, rows_v, gsem, ssem):
                pltpu.make_async_copy(
                    x_hbm.at[src_g.at[j & (2 * G - 1)]], rows_v, gsem).wait()
                sc = pltpu.make_async_copy(
                    rows_v, acc_sh.at[dst_g.at[j & (2 * G - 1)]], ssem)
                sc.start(add=True)
                sc.wait()

                @pl.when(j + 2 < chunks)
                def _():
                    pltpu.make_async_copy(
                        x_hbm.at[src_g.at[(j + 2) & (2 * G - 1)]],
                        rows_v, gsem).start()

            def pair(p, carry):
                j = 2 * p
                g = p // (G // 2)

                @pl.when((p % (G // 2) == 0) & (g + 1 < groups))
                def _():
                    refill(g + 1)
                stage(j, rows0, g0, s0)
                stage(j + 1, rows1, g1, s1)

                @pl.when((p % (G // 2) == G // 2 - 2) & (g + 1 < groups))
                def _():
                    refill_wait()
                return carry
            lax.fori_loop(0, chunks // 2, pair, 0)
            plsc.subcore_barrier()
            for r in range(ROWS_PT // K):
                pltpu.make_async_copy(
                    acc_sh.at[pl.ds(sid * ROWS_PT + r * K, K)],
                    a_hbm.at[pl.ds(sid * ROWS_PT + r * K, K)], s0).start()
            for r in range(ROWS_PT // K):
                pltpu.make_async_copy(
                    acc_sh.at[pl.ds(sid * ROWS_PT + r * K, K)],
                    a_hbm.at[pl.ds(sid * ROWS_PT + r * K, K)], s0).wait()

        @pl.when(cid == 0)
        def _():
            run(x0_hbm, a0_hbm)

        @pl.when(cid == 1)
        def _():
            run(x1_hbm, a1_hbm)

    return agg_kernel(x0, x1, srcp, dstp, zhalf)


def _tc_prep(hp, dsrc, ddst):
    def body(h_ref, ds_ref, dd_ref, rin_ref, rout_ref, xs_ref):
        i = pl.program_id(0)
        rows = i * TN + lax.broadcasted_iota(jnp.int32, (TN, 1), 0)
        valid = (rows < NN).astype(jnp.float32)
        rout = lax.rsqrt(jnp.maximum(ds_ref[...], 1.0)) * valid
        rin = lax.rsqrt(jnp.maximum(dd_ref[...], 1.0)) * valid
        rin_ref[...] = rin
        rout_ref[...] = rout
        xs_ref[...] = h_ref[...] * rout

    f32 = jnp.float32
    return pl.pallas_call(
        body,
        grid=(GRID_N,),
        in_specs=[pl.BlockSpec((TN, 128), lambda i: (i, 0)),
                  pl.BlockSpec((TN, 1), lambda i: (i, 0)),
                  pl.BlockSpec((TN, 1), lambda i: (i, 0))],
        out_specs=[pl.BlockSpec((TN, 1), lambda i: (i, 0)),
                   pl.BlockSpec((TN, 1), lambda i: (i, 0)),
                   pl.BlockSpec((TN, 128), lambda i: (i, 0))],
        out_shape=[jax.ShapeDtypeStruct((N_PAD, 1), f32),
                   jax.ShapeDtypeStruct((N_PAD, 1), f32),
                   jax.ShapeDtypeStruct((N_PAD, 128), f32)],
    )(hp, dsrc, ddst)


def _tc_layer(a0, a1, w0, w1, b, rin, rout):
    def body(a0_ref, a1_ref, w0_ref, w1_ref, b_ref, rin_ref, rout_ref,
             y0_ref, y1_ref):
        t = lax.dot_general(a0_ref[...], w0_ref[...],
                            (((1,), (0,)), ((), ())), **_DOT)
        t += lax.dot_general(a1_ref[...], w1_ref[...],
                             (((1,), (0,)), ((), ())), **_DOT)
        t = jnp.maximum(t * rin_ref[...] + b_ref[...], 0.0) * rout_ref[...]
        y0_ref[...] = t[:, :128]
        y1_ref[...] = t[:, 128:]

    f32 = jnp.float32
    return pl.pallas_call(
        body,
        grid=(GRID_N,),
        in_specs=[pl.BlockSpec((TN, 128), lambda i: (i, 0)),
                  pl.BlockSpec((TN, 128), lambda i: (i, 0)),
                  pl.BlockSpec((128, 256), lambda i: (0, 0)),
                  pl.BlockSpec((128, 256), lambda i: (0, 0)),
                  pl.BlockSpec((1, 256), lambda i: (0, 0)),
                  pl.BlockSpec((TN, 1), lambda i: (i, 0)),
                  pl.BlockSpec((TN, 1), lambda i: (i, 0))],
        out_specs=[pl.BlockSpec((TN, 128), lambda i: (i, 0)),
                   pl.BlockSpec((TN, 128), lambda i: (i, 0))],
        out_shape=[jax.ShapeDtypeStruct((N_PAD, 128), f32),
                   jax.ShapeDtypeStruct((N_PAD, 128), f32)],
    )(a0, a1, w0, w1, b, rin, rout)


def _tc_final(a0, a1, w0, w1, b, rin, wc, bc):
    def body(a0_ref, a1_ref, w0_ref, w1_ref, b_ref, rin_ref, wc_ref, bc_ref,
             o_ref):
        i = pl.program_id(0)
        t = lax.dot_general(a0_ref[...], w0_ref[...],
                            (((1,), (0,)), ((), ())), **_DOT)
        t += lax.dot_general(a1_ref[...], w1_ref[...],
                             (((1,), (0,)), ((), ())), **_DOT)
        t = jnp.maximum(t * rin_ref[...] + b_ref[...], 0.0)
        rows = i * TN + lax.broadcasted_iota(jnp.int32, (TN, 1), 0)
        t = jnp.where(rows < NN, t, 0.0)
        cs = jnp.sum(t, axis=0, keepdims=True)
        part = lax.dot_general(cs, wc_ref[...], (((1,), (0,)), ((), ())),
                               **_DOT)

        @pl.when(i == 0)
        def _():
            o_ref[...] = part + bc_ref[...]

        @pl.when(i > 0)
        def _():
            o_ref[...] = o_ref[...] + part

    return pl.pallas_call(
        body,
        grid=(GRID_N,),
        in_specs=[pl.BlockSpec((TN, 128), lambda i: (i, 0)),
                  pl.BlockSpec((TN, 128), lambda i: (i, 0)),
                  pl.BlockSpec((128, 256), lambda i: (0, 0)),
                  pl.BlockSpec((128, 256), lambda i: (0, 0)),
                  pl.BlockSpec((1, 256), lambda i: (0, 0)),
                  pl.BlockSpec((TN, 1), lambda i: (i, 0)),
                  pl.BlockSpec((256, 10), lambda i: (0, 0)),
                  pl.BlockSpec((1, 10), lambda i: (0, 0))],
        out_specs=pl.BlockSpec((1, 10), lambda i: (0, 0)),
        out_shape=jax.ShapeDtypeStruct((1, 10), jnp.float32),
    )(a0, a1, w0, w1, b, rin, wc, bc)


def kernel(h, edge_index, W1, b1, W2, b2, W3, b3, W4, b4, W5, b5, Wc, bc):
    f32 = jnp.float32
    src = edge_index[0]
    dst = edge_index[1]
    pad_idx = (NN + (jnp.arange(E_PAD - EE, dtype=jnp.int32) % 8))
    srcp = jnp.concatenate([src, pad_idx]).reshape(E_PAD // K, K)
    dstp = jnp.concatenate([dst, pad_idx]).reshape(E_PAD // K, K)
    hp = jnp.pad(h, ((0, N_PAD - NN), (0, 0)))

    zcol = jnp.zeros((N_PAD,), f32)
    z128 = jnp.zeros((K, 128), f32)

    deg_src, deg_dst = _sc_degrees(srcp, dstp, zcol)
    rin, rout, xs = _tc_prep(hp, deg_src.reshape(N_PAD, 1),
                             deg_dst.reshape(N_PAD, 1))

    a0, a1 = _sc_aggregate(xs, xs, srcp, dstp, z128, edge_split=True)
    x0, x1 = _tc_layer(a0, a1, W1, W1, b1.reshape(1, 256), rin, rout)
    for (W, b) in ((W2, b2), (W3, b3), (W4, b4)):
        a0, a1 = _sc_aggregate(x0, x1, srcp, dstp, z128, edge_split=False)
        x0, x1 = _tc_layer(a0, a1, W[:128], W[128:], b.reshape(1, 256),
                           rin, rout)
    a0, a1 = _sc_aggregate(x0, x1, srcp, dstp, z128, edge_split=False)
    return _tc_final(a0, a1, W5[:128], W5[128:], b5.reshape(1, 256),
                     rin, Wc, bc.reshape(1, 10))
